# baseline (speedup 1.0000x reference)
import functools

import jax
import jax.numpy as jnp
from jax import lax
from jax.experimental import pallas as pl
from jax.experimental.pallas import tpu as pltpu
from jax.experimental.pallas import tpu_sc as plsc

_NC = 2
_NS = 16
_L = 16
_NW = _NC * _NS

_NBUF = 4


def _make_pool(B, D, H, HALF):
    HW = H // _NW
    RW = HW // 2
    mesh = plsc.VectorSubcoreMesh(core_axis_name="c", subcore_axis_name="s")

    @functools.partial(
        pl.kernel,
        mesh=mesh,
        out_type=jax.ShapeDtypeStruct((B, D), jnp.float32),
        compiler_params=pltpu.CompilerParams(use_tc_tiling_on_sc=False),
        scratch_types=[
            pltpu.VMEM((HW, HALF), jnp.int32),
            pltpu.VMEM((_NBUF, HALF, D), jnp.float32),
            pltpu.VMEM((RW, D), jnp.float32),
            pltpu.SemaphoreType.DMA,
            pltpu.SemaphoreType.DMA,
            pltpu.SemaphoreType.DMA,
            pltpu.SemaphoreType.DMA,
        ],
    )
    def pool(idx_hbm, table_hbm, out_hbm, idx_v, rows_v, out_v, s0, s1, s2, s3):
        sems = (s0, s1, s2, s3)
        wid = lax.axis_index("s") * _NC + lax.axis_index("c")
        hbase = wid * HW
        pltpu.sync_copy(idx_hbm.at[pl.ds(hbase, HW)], idx_v)

        def start(h, bslot):
            pltpu.async_copy(
                table_hbm.at[idx_v.at[h]], rows_v.at[bslot], sems[bslot]
            )

        def wait(bslot):
            pltpu.make_async_copy(
                table_hbm.at[idx_v.at[0]], rows_v.at[bslot], sems[bslot]
            ).wait()

        def reduce_buf(bslot):
            def inner(i, acc):
                out = list(acc)
                for u in range(4):
                    s = i * 4 + u
                    for v in range(D // _L):
                        out[v] = out[v] + rows_v[bslot, s, pl.ds(v * _L, _L)]
                return tuple(out)

            z = jnp.zeros((_L,), jnp.float32)
            return lax.fori_loop(0, HALF // 4, inner, (z,) * (D // _L))

        for bslot in range(_NBUF):
            start(bslot, bslot)

        def rowpair(i, carry):
            for pair in range(2):
                accs = None
                for halfslot in range(2):
                    bslot = pair * 2 + halfslot
                    h = i * _NBUF + bslot
                    wait(bslot)
                    acc = reduce_buf(bslot)
                    accs = (
                        acc
                        if accs is None
                        else tuple(a + c for a, c in zip(accs, acc))
                    )

                    @pl.when(h + _NBUF < HW)
                    def _():
                        start(h + _NBUF, bslot)

                r = i * 2 + pair
                for v in range(D // _L):
                    out_v[r, pl.ds(v * _L, _L)] = accs[v]
            return carry

        lax.fori_loop(0, HW // _NBUF, rowpair, 0)
        pltpu.sync_copy(out_v, out_hbm.at[pl.ds(wid * RW, RW)])

    return pool


def _head_body(x_ref, w_ref, b_ref, o_ref, *, inv_seq):
    x = x_ref[...]
    w = w_ref[...]
    logits = (
        jnp.dot(x, w, preferred_element_type=jnp.float32) * inv_seq
        + b_ref[...]
    )
    m = jnp.max(logits, axis=1, keepdims=True)
    e = jnp.exp(logits - m)
    lse = m + jnp.log(jnp.sum(e, axis=1, keepdims=True))
    o_ref[...] = logits - lse


def kernel(text, table, W, b):
    B, S = text.shape
    V, D = table.shape
    C = W.shape[0]

    half = S // 2
    half_pad = ((half + 7) // 8) * 8
    idx2 = text.astype(jnp.int32).reshape(B * 2, half)
    idx2 = jnp.pad(idx2, ((0, 0), (0, half_pad - half)))

    pooled_sum = _make_pool(B, D, B * 2, half_pad)(idx2, table)

    head = pl.pallas_call(
        functools.partial(_head_body, inv_seq=1.0 / S),
        out_shape=jax.ShapeDtypeStruct((B, C), jnp.float32),
    )
    return head(pooled_sum, W.T.astype(jnp.float32), b.reshape(1, C))


# speedup vs baseline: 1.0019x; 1.0019x over previous
import functools

import jax
import jax.numpy as jnp
from jax import lax
from jax.experimental import pallas as pl
from jax.experimental.pallas import tpu as pltpu
from jax.experimental.pallas import tpu_sc as plsc

_NC = 2
_NS = 16
_L = 16
_NW = _NC * _NS

_G = 4
_NBUF = 2


def _make_pool(B, D, H, HALF):
    HW = H // _NW
    RW = HW // 2
    BLOCKS = HW // _G
    mesh = plsc.VectorSubcoreMesh(core_axis_name="c", subcore_axis_name="s")

    @functools.partial(
        pl.kernel,
        mesh=mesh,
        out_type=jax.ShapeDtypeStruct((B, D), jnp.float32),
        compiler_params=pltpu.CompilerParams(use_tc_tiling_on_sc=False),
        scratch_types=[
            pltpu.VMEM((BLOCKS, _G * HALF), jnp.int32),
            pltpu.VMEM((_NBUF, _G * HALF, D), jnp.float32),
            pltpu.VMEM((RW, D), jnp.float32),
            pltpu.SemaphoreType.DMA,
            pltpu.SemaphoreType.DMA,
        ],
    )
    def pool(idx_hbm, table_hbm, out_hbm, idx_v, rows_v, out_v, s0, s1):
        sems = (s0, s1)
        wid = lax.axis_index("s") * _NC + lax.axis_index("c")
        pltpu.sync_copy(idx_hbm.at[pl.ds(wid * BLOCKS, BLOCKS)], idx_v)

        def start(blk, slot):
            pltpu.async_copy(
                table_hbm.at[idx_v.at[blk]], rows_v.at[slot], sems[slot]
            )

        def wait(slot):
            pltpu.make_async_copy(
                table_hbm.at[idx_v.at[0]], rows_v.at[slot], sems[slot]
            ).wait()

        def reduce_half(slot, j):
            def inner(i, acc):
                out = list(acc)
                for u in range(4):
                    s = i * 4 + u
                    for v in range(D // _L):
                        out[v] = out[v] + rows_v[slot, j * HALF + s, pl.ds(v * _L, _L)]
                return tuple(out)

            z = jnp.zeros((_L,), jnp.float32)
            return lax.fori_loop(0, HALF // 4, inner, (z,) * (D // _L))

        for slot in range(_NBUF):
            start(slot, slot)

        def body(i, carry):
            for slot in range(_NBUF):
                blk = _NBUF * i + slot
                wait(slot)
                for pair in range(_G // 2):
                    a = reduce_half(slot, 2 * pair)
                    c = reduce_half(slot, 2 * pair + 1)
                    r = _G // 2 * blk + pair
                    for v in range(D // _L):
                        out_v[r, pl.ds(v * _L, _L)] = a[v] + c[v]

                @pl.when(blk + _NBUF < BLOCKS)
                def _():
                    start(blk + _NBUF, slot)
            return carry

        lax.fori_loop(0, BLOCKS // _NBUF, body, 0)
        pltpu.sync_copy(out_v, out_hbm.at[pl.ds(wid * RW, RW)])

    return pool


def _head_body(x_ref, w_ref, b_ref, o_ref, *, inv_seq):
    x = x_ref[...]
    w = w_ref[...]
    logits = (
        jnp.dot(x, w, preferred_element_type=jnp.float32) * inv_seq
        + b_ref[...]
    )
    m = jnp.max(logits, axis=1, keepdims=True)
    e = jnp.exp(logits - m)
    lse = m + jnp.log(jnp.sum(e, axis=1, keepdims=True))
    o_ref[...] = logits - lse


def kernel(text, table, W, b):
    B, S = text.shape
    V, D = table.shape
    C = W.shape[0]

    half = S // 2
    half_pad = ((half + 7) // 8) * 8
    idx2 = text.astype(jnp.int32).reshape(B * 2, half)
    idx2 = jnp.pad(idx2, ((0, 0), (0, half_pad - half)))
    idx3 = idx2.reshape(B * 2 // _G, _G * half_pad)

    pooled_sum = _make_pool(B, D, B * 2, half_pad)(idx3, table)

    head = pl.pallas_call(
        functools.partial(_head_body, inv_seq=1.0 / S),
        out_shape=jax.ShapeDtypeStruct((B, C), jnp.float32),
    )
    return head(pooled_sum, W.T.astype(jnp.float32), b.reshape(1, C))
